# baseline (speedup 1.0000x reference)
import functools

import jax
import jax.numpy as jnp
import numpy as np
from jax import lax
from jax.experimental import pallas as pl
from jax.experimental.pallas import tpu as pltpu
from jax.experimental.pallas import tpu_sc as plsc

N = 1_000_000
KSEL = 512
W = 16
C = 62512
NV = C // 16
SKIP15 = (16 * C - N) // 16
CBUF = C + 16
U = 4
MININT = np.int32(-(2 ** 31))


def _keys_of(v):
    u = lax.bitcast_convert_type(v, jnp.int32)
    m = lax.shift_right_arithmetic(u, 31)
    return lax.bitwise_xor(u, lax.bitwise_or(m, MININT))


def _digit(key, shift):
    d = lax.shift_right_logical(key, shift)
    if shift != 24:
        d = lax.bitwise_and(d, jnp.int32(0xFF))
    return d


def _extract(vec, lane, lane_iota):
    return jnp.max(jnp.where(lane_iota == lane, vec, MININT))


def _sc_select_builder():
    mesh = plsc.VectorSubcoreMesh(
        core_axis_name="c", subcore_axis_name="s", num_cores=1,
        num_subcores=W)

    @functools.partial(
        pl.kernel,
        out_type=jax.ShapeDtypeStruct((16,), jnp.float32),
        mesh=mesh,
        compiler_params=pltpu.CompilerParams(needs_layout_passes=False),
        scratch_types=[
            pltpu.VMEM((CBUF,), jnp.float32),
            pltpu.VMEM((256 * 16 * U,), jnp.int32),
            pltpu.VMEM((256,), jnp.int32),
            pltpu.VMEM((16, 256), jnp.int32),
            pltpu.VMEM((256,), jnp.int32),
            pltpu.VMEM((16,), jnp.float32),
            pltpu.VMEM((16,), jnp.int32),
            pltpu.VMEM_SHARED((W, 256 * 16), jnp.int32),
            pltpu.VMEM_SHARED((256,), jnp.int32),
        ],
    )
    def sc_select(x_hbm, out_hbm, buf, hist, accv, stage, cnts, outv,
                  cvec_v, sh_hist, sh_cnt):
        w = lax.axis_index("s")
        lane = lax.iota(jnp.int32, 16)
        ones = jnp.ones((16,), jnp.int32)
        zero16 = jnp.zeros((16,), jnp.int32)
        base = jnp.where(w == W - 1, jnp.int32(N - C), w * C)
        s0 = jnp.where(w == W - 1, jnp.int32(SKIP15), jnp.int32(0))

        pltpu.sync_copy(x_hbm.at[pl.ds(base, C)], buf.at[pl.ds(0, C)])

        m_cur = jnp.int32(NV) - s0
        v_off = s0
        kk = jnp.int32(KSEL)
        digits = []
        for pidx, shift in enumerate((24, 16, 8, 0)):
            uu = U if pidx == 0 else 1

            def zbody(j, _, hist=hist):
                hist[pl.ds(j * 16, 16)] = zero16
                return 0
            lax.fori_loop(0, 256 * uu, zbody, 0)

            nfull = lax.shift_right_logical(m_cur, 4) if pidx > 0 else m_cur
            if pidx == 0:
                ng = lax.div(nfull - 3, jnp.int32(U))

                def hgroup(g, _):
                    b = v_off + g * U
                    for u in range(U):
                        v = buf[pl.ds((b + u) * 16, 16)]
                        dig = _digit(_keys_of(v), shift)
                        idx = lax.bitwise_or(lax.shift_left(dig, 4), lane)
                        idx = lax.bitwise_or(idx, jnp.int32(u * 4096))
                        plsc.addupdate_scatter(hist, [idx], ones)
                    return 0
                lax.fori_loop(0, ng, hgroup, 0)

                def hrem(i, _):
                    v = buf[pl.ds(i * 16, 16)]
                    dig = _digit(_keys_of(v), shift)
                    idx = lax.bitwise_or(lax.shift_left(dig, 4), lane)
                    plsc.addupdate_scatter(hist, [idx], ones)
                    return 0
                lax.fori_loop(v_off + ng * U, v_off + nfull, hrem, 0)

                def mbody(j, _):
                    s = hist[pl.ds(j * 16, 16)]
                    for u in range(1, U):
                        s = s + hist[pl.ds(u * 4096 + j * 16, 16)]
                    hist[pl.ds(j * 16, 16)] = s
                    return 0
                lax.fori_loop(0, 256, mbody, 0)
            else:
                rem = m_cur - lax.shift_left(nfull, 4)

                def hbody(i, _, shift=shift):
                    v = buf[pl.ds(i * 16, 16)]
                    dig = _digit(_keys_of(v), shift)
                    idx = lax.bitwise_or(lax.shift_left(dig, 4), lane)
                    plsc.addupdate_scatter(hist, [idx], ones)
                    return 0
                lax.fori_loop(0, nfull, hbody, 0)
                vt = buf[pl.ds(nfull * 16, 16)]
                digt = _digit(_keys_of(vt), shift)
                idxt = lax.bitwise_or(lax.shift_left(digt, 4), lane)
                plsc.addupdate_scatter(hist, [idxt], ones, mask=lane < rem)

            pltpu.sync_copy(hist.at[pl.ds(0, 4096)], sh_hist.at[w])
            plsc.subcore_barrier()

            pltpu.sync_copy(sh_hist.at[:, pl.ds(w * 256, 256)], stage)

            def abody(j, _):
                accv[pl.ds(j * 16, 16)] = zero16
                return 0
            lax.fori_loop(0, 16, abody, 0)

            def rbody(v, _):
                def addb(j, _):
                    accv[pl.ds(j * 16, 16)] = (accv[pl.ds(j * 16, 16)]
                                               + stage[v, pl.ds(j * 16, 16)])
                    return 0
                lax.fori_loop(0, 16, addb, 0)
                return 0
            lax.fori_loop(0, W, rbody, 0)

            def cbody(j, cvec):
                t = jnp.sum(accv[pl.ds(j * 16, 16)])
                return jnp.where(lane == j, t, cvec)
            cnt_vec = lax.fori_loop(0, 16, cbody, zero16)
            cvec_v[...] = cnt_vec
            pltpu.sync_copy(cvec_v, sh_cnt.at[pl.ds(w * 16, 16)])
            plsc.subcore_barrier()

            pltpu.sync_copy(sh_cnt, cnts)

            def sbody(j, svec):
                t = jnp.sum(cnts[pl.ds(j * 16, 16)])
                return jnp.where(lane == j, t, svec)
            svec = lax.fori_loop(0, 16, sbody, zero16)
            suf_g = lax.rev(plsc.cumsum(lax.rev(svec, (0,))), (0,))
            jsel = jnp.max(jnp.where(suf_g >= kk, lane, jnp.int32(-1)))
            above_g = (_extract(suf_g, jsel, lane)
                       - _extract(svec, jsel, lane))
            kk2 = kk - above_g
            cg = cnts[pl.ds(jsel * 16, 16)]
            suf_l = lax.rev(plsc.cumsum(lax.rev(cg, (0,))), (0,))
            lsel = jnp.max(jnp.where(suf_l >= kk2, lane, jnp.int32(-1)))
            above_l = (_extract(suf_l, lsel, lane)
                       - _extract(cg, lsel, lane))
            d_sel = jsel * 16 + lsel
            kk = kk2 - above_l
            digits.append(d_sel)

            if shift != 0:
                if pidx == 0:
                    ncomp = v_off + m_cur
                    ngc = lax.div(m_cur - 3, jnp.int32(U))

                    def pgroup(g, off, d_sel=d_sel):
                        b = v_off + g * U
                        for u in range(U):
                            v = buf[pl.ds((b + u) * 16, 16)]
                            dig = _digit(_keys_of(v), 24)
                            match = dig == d_sel
                            mc = jnp.where(match, 1, 0).astype(jnp.int32)
                            inc = plsc.cumsum(mc)
                            plsc.store_scatter(buf, [off + inc - mc], v,
                                               mask=match)
                            off = off + jnp.max(inc)
                        return off
                    off = lax.fori_loop(0, ngc, pgroup, jnp.int32(0))

                    def prem(i, off, d_sel=d_sel):
                        v = buf[pl.ds(i * 16, 16)]
                        dig = _digit(_keys_of(v), 24)
                        match = dig == d_sel
                        mc = jnp.where(match, 1, 0).astype(jnp.int32)
                        inc = plsc.cumsum(mc)
                        plsc.store_scatter(buf, [off + inc - mc], v,
                                           mask=match)
                        return off + jnp.max(inc)
                    m_cur = lax.fori_loop(v_off + ngc * U, ncomp, prem, off)
                    v_off = jnp.int32(0)
                else:
                    nfull2 = lax.shift_right_logical(m_cur + 15, 4)

                    def pbody(i, off, shift=shift, d_sel=d_sel, mp=m_cur):
                        v = buf[pl.ds(i * 16, 16)]
                        dig = _digit(_keys_of(v), shift)
                        valid = (i * 16 + lane) < mp
                        match = jnp.logical_and(dig == d_sel, valid)
                        mc = jnp.where(match, 1, 0).astype(jnp.int32)
                        inc = plsc.cumsum(mc)
                        plsc.store_scatter(buf, [off + inc - mc], v,
                                           mask=match)
                        return off + jnp.max(inc)
                    m_cur = lax.fori_loop(0, nfull2, pbody, jnp.int32(0))

        d0, d1, d2, d3 = digits
        tk = (lax.shift_left(d0, 24) | lax.shift_left(d1, 16)
              | lax.shift_left(d2, 8) | d3)
        tkv = jnp.full((16,), tk, jnp.int32)
        bits = jnp.where(tkv < 0, lax.bitwise_xor(tkv, MININT),
                         lax.bitwise_not(tkv))
        outv[...] = lax.bitcast_convert_type(bits, jnp.float32)

        @pl.when(w == 0)
        def _():
            pltpu.sync_copy(outv, out_hbm)

    return sc_select


_sc_select_cache = []


def _sc_select(x):
    if not _sc_select_cache:
        _sc_select_cache.append(_sc_select_builder())
    return _sc_select_cache[0](x)


def _mask_body(t_ref, x_ref, o_ref):
    t = t_ref[0]
    x = x_ref[...]
    o_ref[...] = jnp.where(x >= t, x, jnp.float32(0.0))


def _mask(t1, x2d):
    return pl.pallas_call(
        _mask_body,
        grid=(5,),
        in_specs=[
            pl.BlockSpec(memory_space=pltpu.SMEM),
            pl.BlockSpec((200, 1000), lambda i: (i, 0)),
        ],
        out_specs=pl.BlockSpec((200, 1000), lambda i: (i, 0)),
        out_shape=jax.ShapeDtypeStruct((1000, 1000), jnp.float32),
    )(t1, x2d)


@jax.jit
def kernel(inputs):
    thr16 = _sc_select(inputs)
    t1 = lax.slice(thr16, (0,), (1,))
    out2d = _mask(t1, inputs.reshape(1000, 1000))
    return out2d.reshape(N)


# speedup vs baseline: 1.1765x; 1.1765x over previous
import functools

import jax
import jax.numpy as jnp
import numpy as np
from jax import lax
from jax.experimental import pallas as pl
from jax.experimental.pallas import tpu as pltpu
from jax.experimental.pallas import tpu_sc as plsc

N = 1_000_000
KSEL = 512
W = 16
C = 62512
NV = C // 16
SKIP15 = (16 * C - N) // 16
CBUF = C + 16
U = 4
MININT = np.int32(-(2 ** 31))


def _keys_of(v):
    u = lax.bitcast_convert_type(v, jnp.int32)
    m = lax.shift_right_arithmetic(u, 31)
    return lax.bitwise_xor(u, lax.bitwise_or(m, MININT))


def _digit(key, shift):
    d = lax.shift_right_logical(key, shift)
    if shift != 24:
        d = lax.bitwise_and(d, jnp.int32(0xFF))
    return d


def _extract(vec, lane, lane_iota):
    return jnp.max(jnp.where(lane_iota == lane, vec, MININT))


def _sc_select_builder():
    mesh = plsc.VectorSubcoreMesh(
        core_axis_name="c", subcore_axis_name="s", num_cores=1,
        num_subcores=W)

    @functools.partial(
        pl.kernel,
        out_type=jax.ShapeDtypeStruct((16,), jnp.float32),
        mesh=mesh,
        compiler_params=pltpu.CompilerParams(needs_layout_passes=False),
        scratch_types=[
            pltpu.VMEM((CBUF,), jnp.float32),
            pltpu.VMEM((256 * 16 * U,), jnp.int32),
            pltpu.VMEM((16, 256), jnp.int32),
            pltpu.VMEM((256,), jnp.int32),
            pltpu.VMEM((16,), jnp.float32),
            pltpu.VMEM((16,), jnp.int32),
            pltpu.VMEM_SHARED((W, 256 * 16), jnp.int32),
            pltpu.VMEM_SHARED((256,), jnp.int32),
        ],
    )
    def sc_select(x_hbm, out_hbm, buf, hist, stage, cnts, outv,
                  cvec_v, sh_hist, sh_cnt):
        w = lax.axis_index("s")
        lane = lax.iota(jnp.int32, 16)
        ones = jnp.ones((16,), jnp.int32)
        zero16 = jnp.zeros((16,), jnp.int32)
        lane_u = [lax.bitwise_or(lane, jnp.int32(u * 4096)) for u in range(U)]
        base = jnp.where(w == W - 1, jnp.int32(N - C), w * C)
        s0 = jnp.where(w == W - 1, jnp.int32(SKIP15), jnp.int32(0))

        pltpu.sync_copy(x_hbm.at[pl.ds(base, C)], buf.at[pl.ds(0, C)])

        m_cur = jnp.int32(NV) - s0
        v_off = s0
        kk = jnp.int32(KSEL)
        digits = []
        for pidx, shift in enumerate((24, 16, 8, 0)):
            uu = U if pidx == 0 else 1

            def zbody(j, _, hist=hist):
                for t in range(8):
                    hist[pl.ds(j * 128 + t * 16, 16)] = zero16
                return 0
            lax.fori_loop(0, 32 * uu, zbody, 0)

            nfull = lax.shift_right_logical(m_cur, 4) if pidx > 0 else m_cur
            if pidx == 0:
                ng = lax.div(nfull - 3, jnp.int32(U))

                def hgroup(g, _):
                    b = v_off + g * U
                    for u in range(U):
                        v = buf[pl.ds((b + u) * 16, 16)]
                        uraw = lax.bitcast_convert_type(v, jnp.int32)
                        d16 = lax.bitwise_and(
                            lax.shift_right_logical(uraw, 20),
                            jnp.int32(0xFF0))
                        idx = lax.bitwise_or(d16, lane_u[u])
                        plsc.addupdate_scatter(hist, [idx], ones)
                    return 0
                lax.fori_loop(0, ng, hgroup, 0)

                def hrem(i, _):
                    v = buf[pl.ds(i * 16, 16)]
                    uraw = lax.bitcast_convert_type(v, jnp.int32)
                    d16 = lax.bitwise_and(
                        lax.shift_right_logical(uraw, 20), jnp.int32(0xFF0))
                    idx = lax.bitwise_or(d16, lane_u[0])
                    plsc.addupdate_scatter(hist, [idx], ones)
                    return 0
                lax.fori_loop(v_off + ng * U, v_off + nfull, hrem, 0)

                def mbody(j, _):
                    for t in range(4):
                        jj = j * 4 + t
                        s = hist[pl.ds(jj * 16, 16)]
                        for u in range(1, U):
                            s = s + hist[pl.ds(u * 4096 + jj * 16, 16)]
                        hist[pl.ds(jj * 16, 16)] = s
                    return 0
                lax.fori_loop(0, 64, mbody, 0)
            else:
                rem = m_cur - lax.shift_left(nfull, 4)

                def hbody(i, _, shift=shift):
                    v = buf[pl.ds(i * 16, 16)]
                    dig = _digit(_keys_of(v), shift)
                    idx = lax.bitwise_or(lax.shift_left(dig, 4), lane)
                    plsc.addupdate_scatter(hist, [idx], ones)
                    return 0
                lax.fori_loop(0, nfull, hbody, 0)
                vt = buf[pl.ds(nfull * 16, 16)]
                digt = _digit(_keys_of(vt), shift)
                idxt = lax.bitwise_or(lax.shift_left(digt, 4), lane)
                plsc.addupdate_scatter(hist, [idxt], ones, mask=lane < rem)

            pltpu.sync_copy(hist.at[pl.ds(0, 4096)], sh_hist.at[w])
            plsc.subcore_barrier()

            pltpu.sync_copy(sh_hist.at[:, pl.ds(w * 256, 256)], stage)

            def rbody(j, cvec):
                s = stage[0, pl.ds(j * 16, 16)]
                for v in range(1, W):
                    s = s + stage[v, pl.ds(j * 16, 16)]
                t = jnp.sum(s)
                return jnp.where(lane == j, t, cvec)
            cnt_vec = lax.fori_loop(0, 16, rbody, zero16)
            cvec_v[...] = cnt_vec
            pltpu.sync_copy(cvec_v, sh_cnt.at[pl.ds(w * 16, 16)])
            plsc.subcore_barrier()

            pltpu.sync_copy(sh_cnt, cnts)

            svec = zero16
            for j in range(16):
                graw = (15 - j if j < 8 else j - 8) if pidx == 0 else j
                t = jnp.sum(cnts[pl.ds(graw * 16, 16)])
                svec = jnp.where(lane == j, t, svec)
            suf_g = lax.rev(plsc.cumsum(lax.rev(svec, (0,))), (0,))
            jsel = jnp.max(jnp.where(suf_g >= kk, lane, jnp.int32(-1)))
            above_g = (_extract(suf_g, jsel, lane)
                       - _extract(svec, jsel, lane))
            kk2 = kk - above_g
            if pidx == 0:
                graw_sel = jnp.where(jsel < 8, 15 - jsel, jsel - 8)
                cg_raw = cnts[pl.ds(graw_sel * 16, 16)]
                condv = (zero16 + jsel) < 8
                cg = jnp.where(condv, lax.rev(cg_raw, (0,)), cg_raw)
            else:
                cg = cnts[pl.ds(jsel * 16, 16)]
            suf_l = lax.rev(plsc.cumsum(lax.rev(cg, (0,))), (0,))
            lsel = jnp.max(jnp.where(suf_l >= kk2, lane, jnp.int32(-1)))
            above_l = (_extract(suf_l, lsel, lane)
                       - _extract(cg, lsel, lane))
            d_sel = jsel * 16 + lsel
            if pidx == 0:
                d_cmp = (graw_sel * 16
                         + jnp.where(jsel < 8, 15 - lsel, lsel))
            else:
                d_cmp = d_sel
            kk = kk2 - above_l
            digits.append(d_sel)

            if shift != 0:
                if pidx == 0:
                    ncomp = v_off + m_cur
                    ngc = lax.div(m_cur - 3, jnp.int32(U))

                    def pgroup(g, off, d_cmp=d_cmp):
                        b = v_off + g * U
                        for u in range(U):
                            v = buf[pl.ds((b + u) * 16, 16)]
                            uraw = lax.bitcast_convert_type(v, jnp.int32)
                            dig = lax.shift_right_logical(uraw, 24)
                            match = dig == d_cmp
                            mc = jnp.where(match, 1, 0).astype(jnp.int32)
                            inc = plsc.cumsum(mc)
                            plsc.store_scatter(buf, [off + inc - mc], v,
                                               mask=match)
                            off = off + jnp.max(inc)
                        return off
                    off = lax.fori_loop(0, ngc, pgroup, jnp.int32(0))

                    def prem(i, off, d_cmp=d_cmp):
                        v = buf[pl.ds(i * 16, 16)]
                        uraw = lax.bitcast_convert_type(v, jnp.int32)
                        dig = lax.shift_right_logical(uraw, 24)
                        match = dig == d_cmp
                        mc = jnp.where(match, 1, 0).astype(jnp.int32)
                        inc = plsc.cumsum(mc)
                        plsc.store_scatter(buf, [off + inc - mc], v,
                                           mask=match)
                        return off + jnp.max(inc)
                    m_cur = lax.fori_loop(v_off + ngc * U, ncomp, prem, off)
                    v_off = jnp.int32(0)
                else:
                    nfull2 = lax.shift_right_logical(m_cur + 15, 4)

                    def pbody(i, off, shift=shift, d_cmp=d_cmp, mp=m_cur):
                        v = buf[pl.ds(i * 16, 16)]
                        dig = _digit(_keys_of(v), shift)
                        valid = (i * 16 + lane) < mp
                        match = jnp.logical_and(dig == d_cmp, valid)
                        mc = jnp.where(match, 1, 0).astype(jnp.int32)
                        inc = plsc.cumsum(mc)
                        plsc.store_scatter(buf, [off + inc - mc], v,
                                           mask=match)
                        return off + jnp.max(inc)
                    m_cur = lax.fori_loop(0, nfull2, pbody, jnp.int32(0))

        d0, d1, d2, d3 = digits
        tk = (lax.shift_left(d0, 24) | lax.shift_left(d1, 16)
              | lax.shift_left(d2, 8) | d3)
        tkv = jnp.full((16,), tk, jnp.int32)
        bits = jnp.where(tkv < 0, lax.bitwise_xor(tkv, MININT),
                         lax.bitwise_not(tkv))
        outv[...] = lax.bitcast_convert_type(bits, jnp.float32)

        @pl.when(w == 0)
        def _():
            pltpu.sync_copy(outv, out_hbm)

    return sc_select


_sc_select_cache = []


def _sc_select(x):
    if not _sc_select_cache:
        _sc_select_cache.append(_sc_select_builder())
    return _sc_select_cache[0](x)


def _mask_body(t_ref, x_ref, o_ref):
    t = t_ref[0]
    x = x_ref[...]
    o_ref[...] = jnp.where(x >= t, x, jnp.float32(0.0))


def _mask(t1, x2d):
    return pl.pallas_call(
        _mask_body,
        grid=(5,),
        in_specs=[
            pl.BlockSpec(memory_space=pltpu.SMEM),
            pl.BlockSpec((200, 1000), lambda i: (i, 0)),
        ],
        out_specs=pl.BlockSpec((200, 1000), lambda i: (i, 0)),
        out_shape=jax.ShapeDtypeStruct((1000, 1000), jnp.float32),
    )(t1, x2d)


@jax.jit
def kernel(inputs):
    thr16 = _sc_select(inputs)
    t1 = lax.slice(thr16, (0,), (1,))
    out2d = _mask(t1, inputs.reshape(1000, 1000))
    return out2d.reshape(N)


# speedup vs baseline: 1.2542x; 1.0660x over previous
import functools

import jax
import jax.numpy as jnp
import numpy as np
from jax import lax
from jax.experimental import pallas as pl
from jax.experimental.pallas import tpu as pltpu
from jax.experimental.pallas import tpu_sc as plsc

N = 1_000_000
KSEL = 512
W = 16
C = 62512
NV = C // 16
SKIP15 = (16 * C - N) // 16
CBUF = C + 16
U = 4
MININT = np.int32(-(2 ** 31))


def _keys_of(v):
    u = lax.bitcast_convert_type(v, jnp.int32)
    m = lax.shift_right_arithmetic(u, 31)
    return lax.bitwise_xor(u, lax.bitwise_or(m, MININT))


def _digit(key, shift):
    d = lax.shift_right_logical(key, shift)
    if shift != 24:
        d = lax.bitwise_and(d, jnp.int32(0xFF))
    return d


def _extract(vec, lane, lane_iota):
    return jnp.max(jnp.where(lane_iota == lane, vec, MININT))


def _sc_select_builder():
    mesh = plsc.VectorSubcoreMesh(
        core_axis_name="c", subcore_axis_name="s", num_cores=1,
        num_subcores=W)

    @functools.partial(
        pl.kernel,
        out_type=jax.ShapeDtypeStruct((16,), jnp.float32),
        mesh=mesh,
        compiler_params=pltpu.CompilerParams(needs_layout_passes=False),
        scratch_types=[
            pltpu.VMEM((CBUF,), jnp.float32),
            pltpu.VMEM((256 * 16 * U,), jnp.int32),
            pltpu.VMEM((16, 256), jnp.int32),
            pltpu.VMEM((256,), jnp.int32),
            pltpu.VMEM((16,), jnp.float32),
            pltpu.VMEM((16,), jnp.int32),
            pltpu.SMEM((1,), jnp.int32),
            pltpu.VMEM_SHARED((W, 256 * 16), jnp.int32),
            pltpu.VMEM_SHARED((256,), jnp.int32),
        ],
    )
    def sc_select(x_hbm, out_hbm, buf, hist, stage, cnts, outv,
                  cvec_v, offm, sh_hist, sh_cnt):
        w = lax.axis_index("s")
        lane = lax.iota(jnp.int32, 16)
        ones = jnp.ones((16,), jnp.int32)
        zero16 = jnp.zeros((16,), jnp.int32)
        lane_u = [lax.bitwise_or(lane, jnp.int32(u * 4096)) for u in range(U)]
        base = jnp.where(w == W - 1, jnp.int32(N - C), w * C)
        s0 = jnp.where(w == W - 1, jnp.int32(SKIP15), jnp.int32(0))

        pltpu.sync_copy(x_hbm.at[pl.ds(base, C)], buf.at[pl.ds(0, C)])

        m_cur = jnp.int32(NV) - s0
        v_off = s0
        kk = jnp.int32(KSEL)
        digits = []
        for pidx, shift in enumerate((24, 16, 8, 0)):
            uu = U if pidx == 0 else 1

            def zbody(j, _, hist=hist):
                for t in range(8):
                    hist[pl.ds(j * 128 + t * 16, 16)] = zero16
                return 0
            lax.fori_loop(0, 32 * uu, zbody, 0)

            nfull = lax.shift_right_logical(m_cur, 4) if pidx > 0 else m_cur
            if pidx == 0:
                ng = lax.div(nfull - 3, jnp.int32(U))

                def hgroup(g, _):
                    b = v_off + g * U
                    for u in range(U):
                        v = buf[pl.ds((b + u) * 16, 16)]
                        uraw = lax.bitcast_convert_type(v, jnp.int32)
                        d16 = lax.bitwise_and(
                            lax.shift_right_logical(uraw, 20),
                            jnp.int32(0xFF0))
                        idx = lax.bitwise_or(d16, lane_u[u])
                        plsc.addupdate_scatter(hist, [idx], ones)
                    return 0
                lax.fori_loop(0, ng, hgroup, 0)

                def hrem(i, _):
                    v = buf[pl.ds(i * 16, 16)]
                    uraw = lax.bitcast_convert_type(v, jnp.int32)
                    d16 = lax.bitwise_and(
                        lax.shift_right_logical(uraw, 20), jnp.int32(0xFF0))
                    idx = lax.bitwise_or(d16, lane_u[0])
                    plsc.addupdate_scatter(hist, [idx], ones)
                    return 0
                lax.fori_loop(v_off + ng * U, v_off + nfull, hrem, 0)

                def mbody(j, _):
                    for t in range(4):
                        jj = j * 4 + t
                        s = hist[pl.ds(jj * 16, 16)]
                        for u in range(1, U):
                            s = s + hist[pl.ds(u * 4096 + jj * 16, 16)]
                        hist[pl.ds(jj * 16, 16)] = s
                    return 0
                lax.fori_loop(0, 64, mbody, 0)
            else:
                rem = m_cur - lax.shift_left(nfull, 4)

                def hbody(i, _, shift=shift):
                    v = buf[pl.ds(i * 16, 16)]
                    dig = _digit(_keys_of(v), shift)
                    idx = lax.bitwise_or(lax.shift_left(dig, 4), lane)
                    plsc.addupdate_scatter(hist, [idx], ones)
                    return 0
                lax.fori_loop(0, nfull, hbody, 0)
                vt = buf[pl.ds(nfull * 16, 16)]
                digt = _digit(_keys_of(vt), shift)
                idxt = lax.bitwise_or(lax.shift_left(digt, 4), lane)
                plsc.addupdate_scatter(hist, [idxt], ones, mask=lane < rem)

            pltpu.sync_copy(hist.at[pl.ds(0, 4096)], sh_hist.at[w])
            plsc.subcore_barrier()

            pltpu.sync_copy(sh_hist.at[:, pl.ds(w * 256, 256)], stage)

            def rbody(j, cvec):
                s = stage[0, pl.ds(j * 16, 16)]
                for v in range(1, W):
                    s = s + stage[v, pl.ds(j * 16, 16)]
                t = jnp.sum(s)
                return jnp.where(lane == j, t, cvec)
            cnt_vec = lax.fori_loop(0, 16, rbody, zero16)
            cvec_v[...] = cnt_vec
            pltpu.sync_copy(cvec_v, sh_cnt.at[pl.ds(w * 16, 16)])
            plsc.subcore_barrier()

            pltpu.sync_copy(sh_cnt, cnts)

            svec = zero16
            for j in range(16):
                graw = (15 - j if j < 8 else j - 8) if pidx == 0 else j
                t = jnp.sum(cnts[pl.ds(graw * 16, 16)])
                svec = jnp.where(lane == j, t, svec)
            suf_g = lax.rev(plsc.cumsum(lax.rev(svec, (0,))), (0,))
            jsel = jnp.max(jnp.where(suf_g >= kk, lane, jnp.int32(-1)))
            above_g = (_extract(suf_g, jsel, lane)
                       - _extract(svec, jsel, lane))
            kk2 = kk - above_g
            if pidx == 0:
                graw_sel = jnp.where(jsel < 8, 15 - jsel, jsel - 8)
                cg_raw = cnts[pl.ds(graw_sel * 16, 16)]
                condv = (zero16 + jsel) < 8
                cg = jnp.where(condv, lax.rev(cg_raw, (0,)), cg_raw)
            else:
                cg = cnts[pl.ds(jsel * 16, 16)]
            suf_l = lax.rev(plsc.cumsum(lax.rev(cg, (0,))), (0,))
            lsel = jnp.max(jnp.where(suf_l >= kk2, lane, jnp.int32(-1)))
            above_l = (_extract(suf_l, lsel, lane)
                       - _extract(cg, lsel, lane))
            d_sel = jsel * 16 + lsel
            if pidx == 0:
                d_cmp = (graw_sel * 16
                         + jnp.where(jsel < 8, 15 - lsel, lsel))
            else:
                d_cmp = d_sel
            kk = kk2 - above_l
            digits.append(d_sel)

            if shift != 0:
                if pidx == 0:
                    ncomp = v_off + m_cur
                    ngc = lax.div(m_cur - 3, jnp.int32(U))
                    offm[0] = jnp.int32(0)

                    def pgroup(g, _, d_cmp=d_cmp):
                        b = v_off + g * U
                        vs, ms = [], []
                        for u in range(U):
                            v = buf[pl.ds((b + u) * 16, 16)]
                            uraw = lax.bitcast_convert_type(v, jnp.int32)
                            dig = lax.shift_right_logical(uraw, 24)
                            vs.append(v)
                            ms.append(dig == d_cmp)
                        anym = ms[0]
                        for u in range(1, U):
                            anym = jnp.logical_or(anym, ms[u])
                        has = jnp.max(jnp.where(anym, 1, 0))

                        @pl.when(has > 0)
                        def _():
                            off = offm[0]
                            for u in range(U):
                                mc = jnp.where(ms[u], 1, 0).astype(jnp.int32)
                                inc = plsc.cumsum(mc)
                                plsc.store_scatter(buf, [off + inc - mc],
                                                   vs[u], mask=ms[u])
                                off = off + jnp.max(inc)
                            offm[0] = off
                        return 0
                    lax.fori_loop(0, ngc, pgroup, 0)

                    def prem(i, _, d_cmp=d_cmp):
                        v = buf[pl.ds(i * 16, 16)]
                        uraw = lax.bitcast_convert_type(v, jnp.int32)
                        dig = lax.shift_right_logical(uraw, 24)
                        match = dig == d_cmp
                        has = jnp.max(jnp.where(match, 1, 0))

                        @pl.when(has > 0)
                        def _():
                            off = offm[0]
                            mc = jnp.where(match, 1, 0).astype(jnp.int32)
                            inc = plsc.cumsum(mc)
                            plsc.store_scatter(buf, [off + inc - mc], v,
                                               mask=match)
                            offm[0] = off + jnp.max(inc)
                        return 0
                    lax.fori_loop(v_off + ngc * U, ncomp, prem, 0)
                    m_cur = offm[0]
                    v_off = jnp.int32(0)
                else:
                    nfull2 = lax.shift_right_logical(m_cur + 15, 4)

                    def pbody(i, off, shift=shift, d_cmp=d_cmp, mp=m_cur):
                        v = buf[pl.ds(i * 16, 16)]
                        dig = _digit(_keys_of(v), shift)
                        valid = (i * 16 + lane) < mp
                        match = jnp.logical_and(dig == d_cmp, valid)
                        mc = jnp.where(match, 1, 0).astype(jnp.int32)
                        inc = plsc.cumsum(mc)
                        plsc.store_scatter(buf, [off + inc - mc], v,
                                           mask=match)
                        return off + jnp.max(inc)
                    m_cur = lax.fori_loop(0, nfull2, pbody, jnp.int32(0))

        d0, d1, d2, d3 = digits
        tk = (lax.shift_left(d0, 24) | lax.shift_left(d1, 16)
              | lax.shift_left(d2, 8) | d3)
        tkv = jnp.full((16,), tk, jnp.int32)
        bits = jnp.where(tkv < 0, lax.bitwise_xor(tkv, MININT),
                         lax.bitwise_not(tkv))
        outv[...] = lax.bitcast_convert_type(bits, jnp.float32)

        @pl.when(w == 0)
        def _():
            pltpu.sync_copy(outv, out_hbm)

    return sc_select


_sc_select_cache = []


def _sc_select(x):
    if not _sc_select_cache:
        _sc_select_cache.append(_sc_select_builder())
    return _sc_select_cache[0](x)


def _mask_body(t_ref, x_ref, o_ref):
    t = t_ref[0]
    x = x_ref[...]
    o_ref[...] = jnp.where(x >= t, x, jnp.float32(0.0))


def _mask(t1, x2d):
    return pl.pallas_call(
        _mask_body,
        grid=(5,),
        in_specs=[
            pl.BlockSpec(memory_space=pltpu.SMEM),
            pl.BlockSpec((200, 1000), lambda i: (i, 0)),
        ],
        out_specs=pl.BlockSpec((200, 1000), lambda i: (i, 0)),
        out_shape=jax.ShapeDtypeStruct((1000, 1000), jnp.float32),
    )(t1, x2d)


@jax.jit
def kernel(inputs):
    thr16 = _sc_select(inputs)
    t1 = lax.slice(thr16, (0,), (1,))
    out2d = _mask(t1, inputs.reshape(1000, 1000))
    return out2d.reshape(N)


# speedup vs baseline: 1.2979x; 1.0349x over previous
import functools

import jax
import jax.numpy as jnp
import numpy as np
from jax import lax
from jax.experimental import pallas as pl
from jax.experimental.pallas import tpu as pltpu
from jax.experimental.pallas import tpu_sc as plsc

N = 1_000_000
KSEL = 512
W = 16
C = 62512
NV = C // 16
SKIP15 = (16 * C - N) // 16
CBUF = C + 16
U = 8
MININT = np.int32(-(2 ** 31))


def _keys_of(v):
    u = lax.bitcast_convert_type(v, jnp.int32)
    m = lax.shift_right_arithmetic(u, 31)
    return lax.bitwise_xor(u, lax.bitwise_or(m, MININT))


def _digit(key, shift):
    d = lax.shift_right_logical(key, shift)
    if shift != 24:
        d = lax.bitwise_and(d, jnp.int32(0xFF))
    return d


def _extract(vec, lane, lane_iota):
    return jnp.max(jnp.where(lane_iota == lane, vec, MININT))


def _sc_select_builder():
    mesh = plsc.VectorSubcoreMesh(
        core_axis_name="c", subcore_axis_name="s", num_cores=1,
        num_subcores=W)

    @functools.partial(
        pl.kernel,
        out_type=jax.ShapeDtypeStruct((16,), jnp.float32),
        mesh=mesh,
        compiler_params=pltpu.CompilerParams(needs_layout_passes=False),
        scratch_types=[
            pltpu.VMEM((CBUF,), jnp.float32),
            pltpu.VMEM((256 * 16 * U,), jnp.int32),
            pltpu.VMEM((16, 256), jnp.int32),
            pltpu.VMEM((256,), jnp.int32),
            pltpu.VMEM((16,), jnp.float32),
            pltpu.VMEM((16,), jnp.int32),
            pltpu.SMEM((1,), jnp.int32),
            pltpu.VMEM_SHARED((W, 256 * 16), jnp.int32),
            pltpu.VMEM_SHARED((256,), jnp.int32),
        ],
    )
    def sc_select(x_hbm, out_hbm, buf, hist, stage, cnts, outv,
                  cvec_v, offm, sh_hist, sh_cnt):
        w = lax.axis_index("s")
        lane = lax.iota(jnp.int32, 16)
        ones = jnp.ones((16,), jnp.int32)
        zero16 = jnp.zeros((16,), jnp.int32)
        lane_u = [lax.bitwise_or(lane, jnp.int32(u * 4096)) for u in range(U)]
        base = jnp.where(w == W - 1, jnp.int32(N - C), w * C)
        s0 = jnp.where(w == W - 1, jnp.int32(SKIP15), jnp.int32(0))

        pltpu.sync_copy(x_hbm.at[pl.ds(base, C)], buf.at[pl.ds(0, C)])

        m_cur = jnp.int32(NV) - s0
        v_off = s0
        kk = jnp.int32(KSEL)
        digits = []
        for pidx, shift in enumerate((24, 16, 8, 0)):
            uu = U if pidx == 0 else 1

            def zbody(j, _, hist=hist):
                for t in range(8):
                    hist[pl.ds(j * 128 + t * 16, 16)] = zero16
                return 0
            lax.fori_loop(0, 32 * uu, zbody, 0)

            nfull = lax.shift_right_logical(m_cur, 4) if pidx > 0 else m_cur
            if pidx == 0:
                ng = lax.div(nfull, jnp.int32(U))

                def hgroup(g, _):
                    b = v_off + g * U
                    for u in range(U):
                        v = buf[pl.ds((b + u) * 16, 16)]
                        uraw = lax.bitcast_convert_type(v, jnp.int32)
                        d16 = lax.bitwise_and(
                            lax.shift_right_logical(uraw, 20),
                            jnp.int32(0xFF0))
                        idx = lax.bitwise_or(d16, lane_u[u])
                        plsc.addupdate_scatter(hist, [idx], ones)
                    return 0
                lax.fori_loop(0, ng, hgroup, 0)

                def hrem(i, _):
                    v = buf[pl.ds(i * 16, 16)]
                    uraw = lax.bitcast_convert_type(v, jnp.int32)
                    d16 = lax.bitwise_and(
                        lax.shift_right_logical(uraw, 20), jnp.int32(0xFF0))
                    idx = lax.bitwise_or(d16, lane_u[0])
                    plsc.addupdate_scatter(hist, [idx], ones)
                    return 0
                lax.fori_loop(v_off + ng * U, v_off + nfull, hrem, 0)

                def mbody(j, _):
                    for t in range(4):
                        jj = j * 4 + t
                        s = hist[pl.ds(jj * 16, 16)]
                        for u in range(1, U):
                            s = s + hist[pl.ds(u * 4096 + jj * 16, 16)]
                        hist[pl.ds(jj * 16, 16)] = s
                    return 0
                lax.fori_loop(0, 64, mbody, 0)
            else:
                rem = m_cur - lax.shift_left(nfull, 4)

                def hbody(i, _, shift=shift):
                    v = buf[pl.ds(i * 16, 16)]
                    dig = _digit(_keys_of(v), shift)
                    idx = lax.bitwise_or(lax.shift_left(dig, 4), lane)
                    plsc.addupdate_scatter(hist, [idx], ones)
                    return 0
                lax.fori_loop(0, nfull, hbody, 0)
                vt = buf[pl.ds(nfull * 16, 16)]
                digt = _digit(_keys_of(vt), shift)
                idxt = lax.bitwise_or(lax.shift_left(digt, 4), lane)
                plsc.addupdate_scatter(hist, [idxt], ones, mask=lane < rem)

            pltpu.sync_copy(hist.at[pl.ds(0, 4096)], sh_hist.at[w])
            plsc.subcore_barrier()

            pltpu.sync_copy(sh_hist.at[:, pl.ds(w * 256, 256)], stage)

            def rbody(j, cvec):
                s = stage[0, pl.ds(j * 16, 16)]
                for v in range(1, W):
                    s = s + stage[v, pl.ds(j * 16, 16)]
                t = jnp.sum(s)
                return jnp.where(lane == j, t, cvec)
            cnt_vec = lax.fori_loop(0, 16, rbody, zero16)
            cvec_v[...] = cnt_vec
            pltpu.sync_copy(cvec_v, sh_cnt.at[pl.ds(w * 16, 16)])
            plsc.subcore_barrier()

            pltpu.sync_copy(sh_cnt, cnts)

            svec = zero16
            for j in range(16):
                graw = (15 - j if j < 8 else j - 8) if pidx == 0 else j
                t = jnp.sum(cnts[pl.ds(graw * 16, 16)])
                svec = jnp.where(lane == j, t, svec)
            suf_g = lax.rev(plsc.cumsum(lax.rev(svec, (0,))), (0,))
            jsel = jnp.max(jnp.where(suf_g >= kk, lane, jnp.int32(-1)))
            above_g = (_extract(suf_g, jsel, lane)
                       - _extract(svec, jsel, lane))
            kk2 = kk - above_g
            if pidx == 0:
                graw_sel = jnp.where(jsel < 8, 15 - jsel, jsel - 8)
                cg_raw = cnts[pl.ds(graw_sel * 16, 16)]
                condv = (zero16 + jsel) < 8
                cg = jnp.where(condv, lax.rev(cg_raw, (0,)), cg_raw)
            else:
                cg = cnts[pl.ds(jsel * 16, 16)]
            suf_l = lax.rev(plsc.cumsum(lax.rev(cg, (0,))), (0,))
            lsel = jnp.max(jnp.where(suf_l >= kk2, lane, jnp.int32(-1)))
            above_l = (_extract(suf_l, lsel, lane)
                       - _extract(cg, lsel, lane))
            d_sel = jsel * 16 + lsel
            if pidx == 0:
                d_cmp = (graw_sel * 16
                         + jnp.where(jsel < 8, 15 - lsel, lsel))
            else:
                d_cmp = d_sel
            kk = kk2 - above_l
            digits.append(d_sel)

            if shift != 0:
                if pidx == 0:
                    ncomp = v_off + m_cur
                    ngc = lax.div(m_cur, jnp.int32(U))
                    offm[0] = jnp.int32(0)

                    def pgroup(g, _, d_cmp=d_cmp):
                        b = v_off + g * U
                        vs, ms = [], []
                        for u in range(U):
                            v = buf[pl.ds((b + u) * 16, 16)]
                            uraw = lax.bitcast_convert_type(v, jnp.int32)
                            dig = lax.shift_right_logical(uraw, 24)
                            vs.append(v)
                            ms.append(dig == d_cmp)
                        anym = ms[0]
                        for u in range(1, U):
                            anym = jnp.logical_or(anym, ms[u])
                        has = jnp.max(jnp.where(anym, 1, 0))

                        @pl.when(has > 0)
                        def _():
                            off = offm[0]
                            for u in range(U):
                                mc = jnp.where(ms[u], 1, 0).astype(jnp.int32)
                                inc = plsc.cumsum(mc)
                                plsc.store_scatter(buf, [off + inc - mc],
                                                   vs[u], mask=ms[u])
                                off = off + jnp.max(inc)
                            offm[0] = off
                        return 0
                    lax.fori_loop(0, ngc, pgroup, 0)

                    def prem(i, _, d_cmp=d_cmp):
                        v = buf[pl.ds(i * 16, 16)]
                        uraw = lax.bitcast_convert_type(v, jnp.int32)
                        dig = lax.shift_right_logical(uraw, 24)
                        match = dig == d_cmp
                        has = jnp.max(jnp.where(match, 1, 0))

                        @pl.when(has > 0)
                        def _():
                            off = offm[0]
                            mc = jnp.where(match, 1, 0).astype(jnp.int32)
                            inc = plsc.cumsum(mc)
                            plsc.store_scatter(buf, [off + inc - mc], v,
                                               mask=match)
                            offm[0] = off + jnp.max(inc)
                        return 0
                    lax.fori_loop(v_off + ngc * U, ncomp, prem, 0)
                    m_cur = offm[0]
                    v_off = jnp.int32(0)
                else:
                    nfull2 = lax.shift_right_logical(m_cur + 15, 4)

                    def pbody(i, off, shift=shift, d_cmp=d_cmp, mp=m_cur):
                        v = buf[pl.ds(i * 16, 16)]
                        dig = _digit(_keys_of(v), shift)
                        valid = (i * 16 + lane) < mp
                        match = jnp.logical_and(dig == d_cmp, valid)
                        mc = jnp.where(match, 1, 0).astype(jnp.int32)
                        inc = plsc.cumsum(mc)
                        plsc.store_scatter(buf, [off + inc - mc], v,
                                           mask=match)
                        return off + jnp.max(inc)
                    m_cur = lax.fori_loop(0, nfull2, pbody, jnp.int32(0))

        d0, d1, d2, d3 = digits
        tk = (lax.shift_left(d0, 24) | lax.shift_left(d1, 16)
              | lax.shift_left(d2, 8) | d3)
        tkv = jnp.full((16,), tk, jnp.int32)
        bits = jnp.where(tkv < 0, lax.bitwise_xor(tkv, MININT),
                         lax.bitwise_not(tkv))
        outv[...] = lax.bitcast_convert_type(bits, jnp.float32)

        @pl.when(w == 0)
        def _():
            pltpu.sync_copy(outv, out_hbm)

    return sc_select


_sc_select_cache = []


def _sc_select(x):
    if not _sc_select_cache:
        _sc_select_cache.append(_sc_select_builder())
    return _sc_select_cache[0](x)


def _mask_body(t_ref, x_ref, o_ref):
    t = t_ref[0]
    x = x_ref[...]
    o_ref[...] = jnp.where(x >= t, x, jnp.float32(0.0))


def _mask(t1, x2d):
    return pl.pallas_call(
        _mask_body,
        grid=(5,),
        in_specs=[
            pl.BlockSpec(memory_space=pltpu.SMEM),
            pl.BlockSpec((200, 1000), lambda i: (i, 0)),
        ],
        out_specs=pl.BlockSpec((200, 1000), lambda i: (i, 0)),
        out_shape=jax.ShapeDtypeStruct((1000, 1000), jnp.float32),
    )(t1, x2d)


@jax.jit
def kernel(inputs):
    thr16 = _sc_select(inputs)
    t1 = lax.slice(thr16, (0,), (1,))
    out2d = _mask(t1, inputs.reshape(1000, 1000))
    return out2d.reshape(N)


# speedup vs baseline: 1.3344x; 1.0281x over previous
import functools

import jax
import jax.numpy as jnp
import numpy as np
from jax import lax
from jax.experimental import pallas as pl
from jax.experimental.pallas import tpu as pltpu
from jax.experimental.pallas import tpu_sc as plsc

N = 1_000_000
KSEL = 512
W = 16
C = 62512
NV = C // 16
SKIP15 = (16 * C - N) // 16
CBUF = C + 16
U = 11
MININT = np.int32(-(2 ** 31))


def _keys_of(v):
    u = lax.bitcast_convert_type(v, jnp.int32)
    m = lax.shift_right_arithmetic(u, 31)
    return lax.bitwise_xor(u, lax.bitwise_or(m, MININT))


def _digit(key, shift):
    d = lax.shift_right_logical(key, shift)
    if shift != 24:
        d = lax.bitwise_and(d, jnp.int32(0xFF))
    return d


def _extract(vec, lane, lane_iota):
    return jnp.max(jnp.where(lane_iota == lane, vec, MININT))


def _sc_select_builder():
    mesh = plsc.VectorSubcoreMesh(
        core_axis_name="c", subcore_axis_name="s", num_cores=1,
        num_subcores=W)

    @functools.partial(
        pl.kernel,
        out_type=jax.ShapeDtypeStruct((16,), jnp.float32),
        mesh=mesh,
        compiler_params=pltpu.CompilerParams(needs_layout_passes=False),
        scratch_types=[
            pltpu.VMEM((CBUF,), jnp.float32),
            pltpu.VMEM((256 * 16 * U,), jnp.int32),
            pltpu.VMEM((16, 256), jnp.int32),
            pltpu.VMEM((256,), jnp.int32),
            pltpu.VMEM((16,), jnp.float32),
            pltpu.VMEM((16,), jnp.int32),
            pltpu.SMEM((1,), jnp.int32),
            pltpu.VMEM_SHARED((W, 256 * 16), jnp.int32),
            pltpu.VMEM_SHARED((256,), jnp.int32),
        ],
    )
    def sc_select(x_hbm, out_hbm, buf, hist, stage, cnts, outv,
                  cvec_v, offm, sh_hist, sh_cnt):
        w = lax.axis_index("s")
        lane = lax.iota(jnp.int32, 16)
        ones = jnp.ones((16,), jnp.int32)
        zero16 = jnp.zeros((16,), jnp.int32)
        lane_u = [lax.bitwise_or(lane, jnp.int32(u * 4096)) for u in range(U)]
        base = jnp.where(w == W - 1, jnp.int32(N - C), w * C)
        s0 = jnp.where(w == W - 1, jnp.int32(SKIP15), jnp.int32(0))

        pltpu.sync_copy(x_hbm.at[pl.ds(base, C)], buf.at[pl.ds(0, C)])

        m_cur = jnp.int32(NV) - s0
        v_off = s0
        kk = jnp.int32(KSEL)
        digits = []
        for pidx, shift in enumerate((24, 16, 8, 0)):
            uu = U if pidx == 0 else 1

            def zbody(j, _, hist=hist):
                for t in range(8):
                    hist[pl.ds(j * 128 + t * 16, 16)] = zero16
                return 0
            lax.fori_loop(0, 32 * uu, zbody, 0)

            nfull = lax.shift_right_logical(m_cur, 4) if pidx > 0 else m_cur
            if pidx == 0:
                ng = lax.div(nfull, jnp.int32(U))

                def hgroup(g, _):
                    b = v_off + g * U
                    for u in range(U):
                        v = buf[pl.ds((b + u) * 16, 16)]
                        uraw = lax.bitcast_convert_type(v, jnp.int32)
                        d16 = lax.bitwise_and(
                            lax.shift_right_logical(uraw, 20),
                            jnp.int32(0xFF0))
                        idx = lax.bitwise_or(d16, lane_u[u])
                        plsc.addupdate_scatter(hist, [idx], ones)
                    return 0
                lax.fori_loop(0, ng, hgroup, 0)

                def hrem(i, _):
                    v = buf[pl.ds(i * 16, 16)]
                    uraw = lax.bitcast_convert_type(v, jnp.int32)
                    d16 = lax.bitwise_and(
                        lax.shift_right_logical(uraw, 20), jnp.int32(0xFF0))
                    idx = lax.bitwise_or(d16, lane_u[0])
                    plsc.addupdate_scatter(hist, [idx], ones)
                    return 0
                lax.fori_loop(v_off + ng * U, v_off + nfull, hrem, 0)

                def mbody(j, _):
                    for t in range(4):
                        jj = j * 4 + t
                        s = hist[pl.ds(jj * 16, 16)]
                        for u in range(1, U):
                            s = s + hist[pl.ds(u * 4096 + jj * 16, 16)]
                        hist[pl.ds(jj * 16, 16)] = s
                    return 0
                lax.fori_loop(0, 64, mbody, 0)
            else:
                rem = m_cur - lax.shift_left(nfull, 4)

                def hbody(i, _, shift=shift):
                    v = buf[pl.ds(i * 16, 16)]
                    dig = _digit(_keys_of(v), shift)
                    idx = lax.bitwise_or(lax.shift_left(dig, 4), lane)
                    plsc.addupdate_scatter(hist, [idx], ones)
                    return 0
                lax.fori_loop(0, nfull, hbody, 0)
                vt = buf[pl.ds(nfull * 16, 16)]
                digt = _digit(_keys_of(vt), shift)
                idxt = lax.bitwise_or(lax.shift_left(digt, 4), lane)
                plsc.addupdate_scatter(hist, [idxt], ones, mask=lane < rem)

            pltpu.sync_copy(hist.at[pl.ds(0, 4096)], sh_hist.at[w])
            plsc.subcore_barrier()

            pltpu.sync_copy(sh_hist.at[:, pl.ds(w * 256, 256)], stage)

            def rbody(j, cvec):
                s = stage[0, pl.ds(j * 16, 16)]
                for v in range(1, W):
                    s = s + stage[v, pl.ds(j * 16, 16)]
                t = jnp.sum(s)
                return jnp.where(lane == j, t, cvec)
            cnt_vec = lax.fori_loop(0, 16, rbody, zero16)
            cvec_v[...] = cnt_vec
            pltpu.sync_copy(cvec_v, sh_cnt.at[pl.ds(w * 16, 16)])
            plsc.subcore_barrier()

            pltpu.sync_copy(sh_cnt, cnts)

            svec = zero16
            for j in range(16):
                graw = (15 - j if j < 8 else j - 8) if pidx == 0 else j
                t = jnp.sum(cnts[pl.ds(graw * 16, 16)])
                svec = jnp.where(lane == j, t, svec)
            suf_g = lax.rev(plsc.cumsum(lax.rev(svec, (0,))), (0,))
            jsel = jnp.max(jnp.where(suf_g >= kk, lane, jnp.int32(-1)))
            above_g = (_extract(suf_g, jsel, lane)
                       - _extract(svec, jsel, lane))
            kk2 = kk - above_g
            if pidx == 0:
                graw_sel = jnp.where(jsel < 8, 15 - jsel, jsel - 8)
                cg_raw = cnts[pl.ds(graw_sel * 16, 16)]
                condv = (zero16 + jsel) < 8
                cg = jnp.where(condv, lax.rev(cg_raw, (0,)), cg_raw)
            else:
                cg = cnts[pl.ds(jsel * 16, 16)]
            suf_l = lax.rev(plsc.cumsum(lax.rev(cg, (0,))), (0,))
            lsel = jnp.max(jnp.where(suf_l >= kk2, lane, jnp.int32(-1)))
            above_l = (_extract(suf_l, lsel, lane)
                       - _extract(cg, lsel, lane))
            d_sel = jsel * 16 + lsel
            if pidx == 0:
                d_cmp = (graw_sel * 16
                         + jnp.where(jsel < 8, 15 - lsel, lsel))
            else:
                d_cmp = d_sel
            kk = kk2 - above_l
            digits.append(d_sel)

            if shift != 0:
                if pidx == 0:
                    ncomp = v_off + m_cur
                    ngc = lax.div(m_cur, jnp.int32(U))
                    offm[0] = jnp.int32(0)

                    def pgroup(g, _, d_cmp=d_cmp):
                        b = v_off + g * U
                        vs, ms = [], []
                        for u in range(U):
                            v = buf[pl.ds((b + u) * 16, 16)]
                            uraw = lax.bitcast_convert_type(v, jnp.int32)
                            dig = lax.shift_right_logical(uraw, 24)
                            vs.append(v)
                            ms.append(dig == d_cmp)
                        anym = ms[0]
                        for u in range(1, U):
                            anym = jnp.logical_or(anym, ms[u])
                        has = jnp.max(jnp.where(anym, 1, 0))

                        @pl.when(has > 0)
                        def _():
                            off = offm[0]
                            for u in range(U):
                                mc = jnp.where(ms[u], 1, 0).astype(jnp.int32)
                                inc = plsc.cumsum(mc)
                                plsc.store_scatter(buf, [off + inc - mc],
                                                   vs[u], mask=ms[u])
                                off = off + jnp.max(inc)
                            offm[0] = off
                        return 0
                    lax.fori_loop(0, ngc, pgroup, 0)

                    def prem(i, _, d_cmp=d_cmp):
                        v = buf[pl.ds(i * 16, 16)]
                        uraw = lax.bitcast_convert_type(v, jnp.int32)
                        dig = lax.shift_right_logical(uraw, 24)
                        match = dig == d_cmp
                        has = jnp.max(jnp.where(match, 1, 0))

                        @pl.when(has > 0)
                        def _():
                            off = offm[0]
                            mc = jnp.where(match, 1, 0).astype(jnp.int32)
                            inc = plsc.cumsum(mc)
                            plsc.store_scatter(buf, [off + inc - mc], v,
                                               mask=match)
                            offm[0] = off + jnp.max(inc)
                        return 0
                    lax.fori_loop(v_off + ngc * U, ncomp, prem, 0)
                    m_cur = offm[0]
                    v_off = jnp.int32(0)
                else:
                    nfull2 = lax.shift_right_logical(m_cur + 15, 4)

                    def pbody(i, off, shift=shift, d_cmp=d_cmp, mp=m_cur):
                        v = buf[pl.ds(i * 16, 16)]
                        dig = _digit(_keys_of(v), shift)
                        valid = (i * 16 + lane) < mp
                        match = jnp.logical_and(dig == d_cmp, valid)
                        mc = jnp.where(match, 1, 0).astype(jnp.int32)
                        inc = plsc.cumsum(mc)
                        plsc.store_scatter(buf, [off + inc - mc], v,
                                           mask=match)
                        return off + jnp.max(inc)
                    m_cur = lax.fori_loop(0, nfull2, pbody, jnp.int32(0))

        d0, d1, d2, d3 = digits
        tk = (lax.shift_left(d0, 24) | lax.shift_left(d1, 16)
              | lax.shift_left(d2, 8) | d3)
        tkv = jnp.full((16,), tk, jnp.int32)
        bits = jnp.where(tkv < 0, lax.bitwise_xor(tkv, MININT),
                         lax.bitwise_not(tkv))
        outv[...] = lax.bitcast_convert_type(bits, jnp.float32)

        @pl.when(w == 0)
        def _():
            pltpu.sync_copy(outv, out_hbm)

    return sc_select


_sc_select_cache = []


def _sc_select(x):
    if not _sc_select_cache:
        _sc_select_cache.append(_sc_select_builder())
    return _sc_select_cache[0](x)


def _mask_body(t_ref, x_ref, o_ref):
    t = t_ref[0]
    x = x_ref[...]
    o_ref[...] = jnp.where(x >= t, x, jnp.float32(0.0))


def _mask(t1, x2d):
    return pl.pallas_call(
        _mask_body,
        grid=(5,),
        in_specs=[
            pl.BlockSpec(memory_space=pltpu.SMEM),
            pl.BlockSpec((200, 1000), lambda i: (i, 0)),
        ],
        out_specs=pl.BlockSpec((200, 1000), lambda i: (i, 0)),
        out_shape=jax.ShapeDtypeStruct((1000, 1000), jnp.float32),
    )(t1, x2d)


@jax.jit
def kernel(inputs):
    thr16 = _sc_select(inputs)
    t1 = lax.slice(thr16, (0,), (1,))
    out2d = _mask(t1, inputs.reshape(1000, 1000))
    return out2d.reshape(N)
